# baseline (speedup 1.0000x reference)
import functools

import jax
import jax.numpy as jnp
from jax import lax
from jax.experimental import pallas as pl
from jax.experimental.pallas import tpu as pltpu
from jax.experimental.pallas import tpu_sc as plsc

N = 10000
E = 320000
D = 128
DE = 16
HID = 128

NC = 2
NS = 16
NW = NC * NS
CW = 80
RW = E // NW
JW = RW // CW
NP = 10240
NTP = NP // NS

BE = 2000
GE = E // BE


def _sc_mesh():
    return plsc.VectorSubcoreMesh(
        core_axis_name="c", subcore_axis_name="s",
        num_cores=NC, num_subcores=NS)


@functools.partial(
    pl.kernel,
    out_type=jax.ShapeDtypeStruct((E, HID), jnp.float32),
    mesh=_sc_mesh(),
    scratch_types=[
        pltpu.VMEM((RW,), jnp.int32),
        pltpu.VMEM((CW, HID), jnp.float32),
        pltpu.VMEM((CW, HID), jnp.float32),
        pltpu.SemaphoreType.DMA,
        pltpu.SemaphoreType.DMA,
        pltpu.SemaphoreType.DMA,
        pltpu.SemaphoreType.DMA,
    ],
)
def _sc_gather(tab_hbm, idx_hbm, out_hbm, idx_v, g0, g1, sg0, sg1, sw0, sw1):
    w = lax.axis_index("s") * NC + lax.axis_index("c")
    pltpu.sync_copy(idx_hbm.at[pl.ds(w * RW, RW)], idx_v)
    gb = (g0, g1)
    sg = (sg0, sg1)
    sw = (sw0, sw1)

    def gath(jj, b):
        off = pl.multiple_of(jj * CW, 16)
        pltpu.async_copy(tab_hbm.at[idx_v.at[pl.ds(off, CW)]], gb[b], sg[b])

    def put(jj, b):
        pltpu.async_copy(gb[b], out_hbm.at[pl.ds(w * RW + jj * CW, CW)], sw[b])

    gath(0, 0)

    def step(i, carry):
        for b in (0, 1):
            jj = 2 * i + b
            nb = 1 - b

            @pl.when(jj >= 1)
            def _():
                pltpu.make_async_copy(gb[nb], out_hbm.at[pl.ds(0, CW)],
                                      sw[nb]).wait()

            gath(jj + 1, nb)
            pltpu.make_async_copy(tab_hbm.at[idx_v.at[pl.ds(0, CW)]], gb[b],
                                  sg[b]).wait()
            put(jj, b)
        return carry

    lax.fori_loop(0, (JW - 1) // 2, step, 0)
    pltpu.make_async_copy(gb[1], out_hbm.at[pl.ds(0, CW)], sw[1]).wait()
    pltpu.make_async_copy(tab_hbm.at[idx_v.at[pl.ds(0, CW)]], gb[0], sg[0]).wait()
    pltpu.sync_copy(gb[0], out_hbm.at[pl.ds(w * RW + (JW - 1) * CW, CW)])


@functools.partial(
    pl.kernel,
    out_type=jax.ShapeDtypeStruct((E, HID), jnp.float32),
    mesh=_sc_mesh(),
    scratch_types=[
        pltpu.VMEM((RW,), jnp.int32),
        pltpu.VMEM((RW,), jnp.int32),
        pltpu.VMEM((HID,), jnp.float32),
        pltpu.VMEM((CW, HID), jnp.float32),
        pltpu.VMEM((CW, HID), jnp.float32),
        pltpu.VMEM((CW, HID), jnp.float32),
        pltpu.VMEM((CW, HID), jnp.float32),
        pltpu.VMEM((CW, HID), jnp.float32),
        pltpu.VMEM((CW, HID), jnp.float32),
        pltpu.SemaphoreType.DMA,
        pltpu.SemaphoreType.DMA,
        pltpu.SemaphoreType.DMA,
        pltpu.SemaphoreType.DMA,
        pltpu.SemaphoreType.DMA,
        pltpu.SemaphoreType.DMA,
        pltpu.SemaphoreType.DMA,
        pltpu.SemaphoreType.DMA,
    ],
)
def _sc_msg_update(s_hbm, k_hbm, h0_hbm, src_hbm, rev_hbm, bh_hbm, out_hbm,
                   isrc_v, irev_v, bh_v, g0, g1, r0, r1, h0b0, h0b1,
                   sg0, sg1, sr0, sr1, sh0, sh1, sw0, sw1):
    w = lax.axis_index("s") * NC + lax.axis_index("c")
    pltpu.sync_copy(src_hbm.at[pl.ds(w * RW, RW)], isrc_v)
    pltpu.sync_copy(rev_hbm.at[pl.ds(w * RW, RW)], irev_v)
    pltpu.sync_copy(bh_hbm, bh_v)
    gb = (g0, g1)
    rb = (r0, r1)
    hb = (h0b0, h0b1)
    sg = (sg0, sg1)
    sr = (sr0, sr1)
    sh = (sh0, sh1)
    sw = (sw0, sw1)

    def gath(jj, b):
        off = pl.multiple_of(jj * CW, 16)
        pltpu.async_copy(s_hbm.at[isrc_v.at[pl.ds(off, CW)]], gb[b], sg[b])
        pltpu.async_copy(k_hbm.at[irev_v.at[pl.ds(off, CW)]], rb[b], sr[b])
        pltpu.async_copy(h0_hbm.at[pl.ds(w * RW + jj * CW, CW)], hb[b], sh[b])

    def wait_gath(b):
        pltpu.make_async_copy(s_hbm.at[isrc_v.at[pl.ds(0, CW)]], gb[b],
                              sg[b]).wait()
        pltpu.make_async_copy(k_hbm.at[irev_v.at[pl.ds(0, CW)]], rb[b],
                              sr[b]).wait()
        pltpu.make_async_copy(h0_hbm.at[pl.ds(0, CW)], hb[b], sh[b]).wait()

    def update(b):
        g, r, h = gb[b], rb[b], hb[b]

        def row(i, carry):
            for t in range(HID // 16):
                d = pl.ds(t * 16, 16)
                v = h[i, d] + g[i, d] - r[i, d] + bh_v[d]
                g[i, d] = jnp.maximum(v, 0.0)
            return carry

        lax.fori_loop(0, CW, row, 0)

    gath(0, 0)

    def step(i, carry):
        for b in (0, 1):
            jj = 2 * i + b
            nb = 1 - b

            @pl.when(jj >= 1)
            def _():
                pltpu.make_async_copy(gb[nb], out_hbm.at[pl.ds(0, CW)],
                                      sw[nb]).wait()

            gath(jj + 1, nb)
            wait_gath(b)
            update(b)
            pltpu.async_copy(gb[b], out_hbm.at[pl.ds(w * RW + jj * CW, CW)],
                             sw[b])
        return carry

    lax.fori_loop(0, (JW - 1) // 2, step, 0)
    pltpu.make_async_copy(gb[1], out_hbm.at[pl.ds(0, CW)], sw[1]).wait()
    wait_gath(0)
    update(0)
    pltpu.sync_copy(gb[0], out_hbm.at[pl.ds(w * RW + (JW - 1) * CW, CW)])


@functools.partial(
    pl.kernel,
    out_type=jax.ShapeDtypeStruct((NC, NP, HID), jnp.float32),
    mesh=_sc_mesh(),
    scratch_types=[
        pltpu.VMEM((RW,), jnp.int32),
        pltpu.VMEM((CW,), jnp.int32),
        pltpu.VMEM((CW,), jnp.int32),
        pltpu.VMEM((CW, HID), jnp.float32),
        pltpu.VMEM((CW, HID), jnp.float32),
        pltpu.VMEM_SHARED((NP, HID), jnp.float32),
        pltpu.SemaphoreType.DMA,
        pltpu.SemaphoreType.DMA,
    ],
)
def _sc_scatter(rows_hbm, dst_hbm, zero_hbm, out_hbm, idx_v, ci0, ci1,
                rv0, rv1, acc_sh, sl0, sl1):
    c = lax.axis_index("c")
    s = lax.axis_index("s")
    w = s * NC + c
    pltpu.sync_copy(zero_hbm.at[pl.ds(s * NTP, NTP)],
                    acc_sh.at[pl.ds(s * NTP, NTP)])
    pltpu.sync_copy(dst_hbm.at[pl.ds(w * RW, RW)], idx_v)
    plsc.subcore_barrier()
    rv = (rv0, rv1)
    ci = (ci0, ci1)
    sl = (sl0, sl1)

    def load(jj, b):
        pltpu.async_copy(rows_hbm.at[pl.ds(w * RW + jj * CW, CW)], rv[b],
                        sl[b])
        off = pl.multiple_of(jj * CW, 16)
        for t in range(CW // 16):
            ci[b][pl.ds(t * 16, 16)] = idx_v[pl.ds(off + t * 16, 16)]

    def scat(b):
        pltpu.make_async_copy(rows_hbm.at[pl.ds(0, CW)], rv[b], sl[b]).wait()
        pltpu.sync_copy(rv[b], acc_sh.at[ci[b]], add=True)

    load(0, 0)

    def step(i, carry):
        for b in (0, 1):
            jj = 2 * i + b
            load(jj + 1, 1 - b)
            scat(b)
        return carry

    lax.fori_loop(0, (JW - 1) // 2, step, 0)
    scat(0)
    plsc.subcore_barrier()
    pltpu.sync_copy(acc_sh.at[pl.ds(s * NTP, NTP)],
                    out_hbm.at[c, pl.ds(s * NTP, NTP)])


def _pre_body(x_ref, w_ref, o_ref):
    o_ref[:] = jnp.dot(x_ref[:], w_ref[:], preferred_element_type=jnp.float32)


_tc_pre = pl.pallas_call(
    _pre_body,
    out_shape=jax.ShapeDtypeStruct((N, HID), jnp.float32),
)


def _edge0_body(xg, ea, wib, bi, wh, h0_ref, k_ref):
    h0 = xg[:] + jnp.dot(ea[:], wib[:], preferred_element_type=jnp.float32) + bi[:]
    h0_ref[:] = h0
    k_ref[:] = jnp.dot(jnp.maximum(h0, 0.0), wh[:],
                       preferred_element_type=jnp.float32)


_tc_edge0 = pl.pallas_call(
    _edge0_body,
    grid=(GE,),
    in_specs=[
        pl.BlockSpec((BE, HID), lambda i: (i, 0)),
        pl.BlockSpec((BE, DE), lambda i: (i, 0)),
        pl.BlockSpec((DE, HID), lambda i: (0, 0)),
        pl.BlockSpec((1, HID), lambda i: (0, 0)),
        pl.BlockSpec((HID, HID), lambda i: (0, 0)),
    ],
    out_specs=[pl.BlockSpec((BE, HID), lambda i: (i, 0))] * 2,
    out_shape=[jax.ShapeDtypeStruct((E, HID), jnp.float32)] * 2,
)


def _comb_body(p_ref, s_ref):
    s_ref[:] = p_ref[0, :N] + p_ref[1, :N]


_tc_combine = pl.pallas_call(
    _comb_body,
    out_shape=jax.ShapeDtypeStruct((N, HID), jnp.float32),
)


def _mm_body(ht, wh, k_ref):
    k_ref[:] = jnp.dot(ht[:], wh[:], preferred_element_type=jnp.float32)


_tc_mm = pl.pallas_call(
    _mm_body,
    grid=(GE,),
    in_specs=[
        pl.BlockSpec((BE, HID), lambda i: (i, 0)),
        pl.BlockSpec((HID, HID), lambda i: (0, 0)),
    ],
    out_specs=pl.BlockSpec((BE, HID), lambda i: (i, 0)),
    out_shape=jax.ShapeDtypeStruct((E, HID), jnp.float32),
)


def _final_body(p, x, woa, wob, bo, o_ref):
    sfull = p[0, :N] + p[1, :N]
    rs = jnp.sum(sfull, axis=1, keepdims=True)
    m = jnp.where(rs == 0.0, x[:], sfull)
    o_ref[:] = jnp.maximum(
        jnp.dot(x[:], woa[:], preferred_element_type=jnp.float32)
        + jnp.dot(m, wob[:], preferred_element_type=jnp.float32) + bo[:], 0.0)


_tc_final = pl.pallas_call(
    _final_body,
    out_shape=jax.ShapeDtypeStruct((N, HID), jnp.float32),
)


def kernel(x, edge_index, edge_attr, rev_edge_index, W_i, b_i, W_h, b_h,
           W_o, b_o):
    src = edge_index[0]
    dst = edge_index[1]
    rev = rev_edge_index
    zeros = jnp.zeros((NP, HID), jnp.float32)
    wia, wib = W_i[:D], W_i[D:]
    woa, wob = W_o[:D], W_o[D:]
    bi = b_i.reshape(1, HID)
    bh = b_h.reshape(1, HID)
    bo = b_o.reshape(1, HID)

    xw = _tc_pre(x, wia)
    xg = _sc_gather(xw, src)
    h0, k = _tc_edge0(xg, edge_attr, wib, bi, W_h)

    ht = None
    for t in range(2):
        p = _sc_scatter(k, dst, zeros)
        s = _tc_combine(p)
        ht = _sc_msg_update(s, k, h0, src, rev, b_h)
        if t == 0:
            k = _tc_mm(ht, W_h)

    p = _sc_scatter(ht, dst, zeros)
    return _tc_final(p, x, woa, wob, bo)


# speedup vs baseline: 1.2449x; 1.2449x over previous
import functools

import jax
import jax.numpy as jnp
from jax import lax
from jax.experimental import pallas as pl
from jax.experimental.pallas import tpu as pltpu
from jax.experimental.pallas import tpu_sc as plsc

N = 10000
E = 320000
D = 128
DE = 16
HID = 128

NC = 2
NS = 16
NW = NC * NS
CW = 80
RW = E // NW
JW = RW // CW
NP = 10240
NTP = NP // NS

BE = 2000
GE = E // BE


def _sc_mesh():
    return plsc.VectorSubcoreMesh(
        core_axis_name="c", subcore_axis_name="s",
        num_cores=NC, num_subcores=NS)


NB = 5


@functools.partial(
    pl.kernel,
    out_type=jax.ShapeDtypeStruct((E, HID), jnp.float32),
    mesh=_sc_mesh(),
    scratch_types=(
        [pltpu.VMEM((RW,), jnp.int32)]
        + [pltpu.VMEM((CW, HID), jnp.float32)] * NB
        + [pltpu.SemaphoreType.DMA] * (2 * NB)
    ),
)
def _sc_gather(tab_hbm, idx_hbm, out_hbm, idx_v, *bufs):
    gb = bufs[:NB]
    sg = bufs[NB:2 * NB]
    sw = bufs[2 * NB:3 * NB]
    w = lax.axis_index("s") * NC + lax.axis_index("c")
    pltpu.sync_copy(idx_hbm.at[pl.ds(w * RW, RW)], idx_v)

    def gath(jj, b):
        off = pl.multiple_of(jj * CW, 16)
        pltpu.async_copy(tab_hbm.at[idx_v.at[pl.ds(off, CW)]], gb[b], sg[b])

    gath(0, 0)

    def step(i, carry):
        for b in range(NB):
            jj = NB * i + b
            nxb = (b + 1) % NB

            @pl.when(jj >= NB - 1)
            def _():
                pltpu.make_async_copy(gb[nxb], out_hbm.at[pl.ds(0, CW)],
                                      sw[nxb]).wait()

            @pl.when(jj < JW - 1)
            def _():
                gath(jj + 1, nxb)

            pltpu.make_async_copy(tab_hbm.at[idx_v.at[pl.ds(0, CW)]], gb[b],
                                  sg[b]).wait()
            pltpu.async_copy(gb[b], out_hbm.at[pl.ds(w * RW + jj * CW, CW)],
                             sw[b])
        return carry

    lax.fori_loop(0, JW // NB, step, 0)
    for b in range(1, NB):
        pltpu.make_async_copy(gb[b], out_hbm.at[pl.ds(0, CW)], sw[b]).wait()


@functools.partial(
    pl.kernel,
    out_type=jax.ShapeDtypeStruct((E, HID), jnp.float32),
    mesh=_sc_mesh(),
    scratch_types=(
        [pltpu.VMEM((RW,), jnp.int32)] * 2
        + [pltpu.VMEM((CW, HID), jnp.float32)] * (2 * NB)
        + [pltpu.SemaphoreType.DMA] * (3 * NB)
    ),
)
def _sc_gather_sub(s_hbm, k_hbm, src_hbm, rev_hbm, out_hbm, isrc_v, irev_v,
                   *bufs):
    gb = bufs[:NB]
    rb = bufs[NB:2 * NB]
    sg = bufs[2 * NB:3 * NB]
    sr = bufs[3 * NB:4 * NB]
    sw = bufs[4 * NB:5 * NB]
    w = lax.axis_index("s") * NC + lax.axis_index("c")
    pltpu.sync_copy(src_hbm.at[pl.ds(w * RW, RW)], isrc_v)
    pltpu.sync_copy(rev_hbm.at[pl.ds(w * RW, RW)], irev_v)

    def gath(jj, b):
        off = pl.multiple_of(jj * CW, 16)
        pltpu.async_copy(s_hbm.at[isrc_v.at[pl.ds(off, CW)]], gb[b], sg[b])
        pltpu.async_copy(k_hbm.at[irev_v.at[pl.ds(off, CW)]], rb[b], sr[b])

    def wait_gath(b):
        pltpu.make_async_copy(s_hbm.at[isrc_v.at[pl.ds(0, CW)]], gb[b],
                              sg[b]).wait()
        pltpu.make_async_copy(k_hbm.at[irev_v.at[pl.ds(0, CW)]], rb[b],
                              sr[b]).wait()

    def sub(b):
        g, r = gb[b], rb[b]

        def row(i, carry):
            for t in range(HID // 16):
                d = pl.ds(t * 16, 16)
                g[i, d] = g[i, d] - r[i, d]
            return carry

        lax.fori_loop(0, CW, row, 0)

    gath(0, 0)

    def step(i, carry):
        for b in range(NB):
            jj = NB * i + b
            nxb = (b + 1) % NB

            @pl.when(jj >= NB - 1)
            def _():
                pltpu.make_async_copy(gb[nxb], out_hbm.at[pl.ds(0, CW)],
                                      sw[nxb]).wait()

            @pl.when(jj < JW - 1)
            def _():
                gath(jj + 1, nxb)

            wait_gath(b)
            sub(b)
            pltpu.async_copy(gb[b], out_hbm.at[pl.ds(w * RW + jj * CW, CW)],
                             sw[b])
        return carry

    lax.fori_loop(0, JW // NB, step, 0)
    for b in range(1, NB):
        pltpu.make_async_copy(gb[b], out_hbm.at[pl.ds(0, CW)], sw[b]).wait()


@functools.partial(
    pl.kernel,
    out_type=jax.ShapeDtypeStruct((NC, NP, HID), jnp.float32),
    mesh=_sc_mesh(),
    scratch_types=[
        pltpu.VMEM((RW,), jnp.int32),
        pltpu.VMEM((CW,), jnp.int32),
        pltpu.VMEM((CW,), jnp.int32),
        pltpu.VMEM((CW, HID), jnp.float32),
        pltpu.VMEM((CW, HID), jnp.float32),
        pltpu.VMEM_SHARED((NP, HID), jnp.float32),
        pltpu.SemaphoreType.DMA,
        pltpu.SemaphoreType.DMA,
    ],
)
def _sc_scatter(rows_hbm, dst_hbm, zero_hbm, out_hbm, idx_v, ci0, ci1,
                rv0, rv1, acc_sh, sl0, sl1):
    c = lax.axis_index("c")
    s = lax.axis_index("s")
    w = s * NC + c
    pltpu.sync_copy(zero_hbm.at[pl.ds(s * NTP, NTP)],
                    acc_sh.at[pl.ds(s * NTP, NTP)])
    pltpu.sync_copy(dst_hbm.at[pl.ds(w * RW, RW)], idx_v)
    plsc.subcore_barrier()
    rv = (rv0, rv1)
    ci = (ci0, ci1)
    sl = (sl0, sl1)

    def load(jj, b):
        pltpu.async_copy(rows_hbm.at[pl.ds(w * RW + jj * CW, CW)], rv[b],
                        sl[b])
        off = pl.multiple_of(jj * CW, 16)
        for t in range(CW // 16):
            ci[b][pl.ds(t * 16, 16)] = idx_v[pl.ds(off + t * 16, 16)]

    def scat(b):
        pltpu.make_async_copy(rows_hbm.at[pl.ds(0, CW)], rv[b], sl[b]).wait()
        pltpu.sync_copy(rv[b], acc_sh.at[ci[b]], add=True)

    load(0, 0)

    def step(i, carry):
        for b in (0, 1):
            jj = 2 * i + b
            load(jj + 1, 1 - b)
            scat(b)
        return carry

    lax.fori_loop(0, (JW - 1) // 2, step, 0)
    scat(0)
    plsc.subcore_barrier()
    pltpu.sync_copy(acc_sh.at[pl.ds(s * NTP, NTP)],
                    out_hbm.at[c, pl.ds(s * NTP, NTP)])


def _pre_body(x_ref, w_ref, o_ref):
    o_ref[:] = jnp.dot(x_ref[:], w_ref[:], preferred_element_type=jnp.float32)


_tc_pre = pl.pallas_call(
    _pre_body,
    out_shape=jax.ShapeDtypeStruct((N, HID), jnp.float32),
)


def _edge0_body(xg, ea, wib, bi, wh, h0_ref, k_ref):
    h0 = xg[:] + jnp.dot(ea[:], wib[:], preferred_element_type=jnp.float32) + bi[:]
    h0_ref[:] = h0
    k_ref[:] = jnp.dot(jnp.maximum(h0, 0.0), wh[:],
                       preferred_element_type=jnp.float32)


_tc_edge0 = pl.pallas_call(
    _edge0_body,
    grid=(GE,),
    in_specs=[
        pl.BlockSpec((BE, HID), lambda i: (i, 0)),
        pl.BlockSpec((BE, DE), lambda i: (i, 0)),
        pl.BlockSpec((DE, HID), lambda i: (0, 0)),
        pl.BlockSpec((1, HID), lambda i: (0, 0)),
        pl.BlockSpec((HID, HID), lambda i: (0, 0)),
    ],
    out_specs=[pl.BlockSpec((BE, HID), lambda i: (i, 0))] * 2,
    out_shape=[jax.ShapeDtypeStruct((E, HID), jnp.float32)] * 2,
)


def _comb_body(p_ref, s_ref):
    s_ref[:] = p_ref[0, :N] + p_ref[1, :N]


_tc_combine = pl.pallas_call(
    _comb_body,
    out_shape=jax.ShapeDtypeStruct((N, HID), jnp.float32),
)


def _mid_body(h0, m, bh, wh, k_ref):
    ht = jnp.maximum(h0[:] + m[:] + bh[:], 0.0)
    k_ref[:] = jnp.dot(ht, wh[:], preferred_element_type=jnp.float32)


_tc_mid = pl.pallas_call(
    _mid_body,
    grid=(GE,),
    in_specs=[
        pl.BlockSpec((BE, HID), lambda i: (i, 0)),
        pl.BlockSpec((BE, HID), lambda i: (i, 0)),
        pl.BlockSpec((1, HID), lambda i: (0, 0)),
        pl.BlockSpec((HID, HID), lambda i: (0, 0)),
    ],
    out_specs=pl.BlockSpec((BE, HID), lambda i: (i, 0)),
    out_shape=jax.ShapeDtypeStruct((E, HID), jnp.float32),
)


def _last_body(h0, m, bh, ht_ref):
    ht_ref[:] = jnp.maximum(h0[:] + m[:] + bh[:], 0.0)


_tc_last = pl.pallas_call(
    _last_body,
    grid=(GE,),
    in_specs=[
        pl.BlockSpec((BE, HID), lambda i: (i, 0)),
        pl.BlockSpec((BE, HID), lambda i: (i, 0)),
        pl.BlockSpec((1, HID), lambda i: (0, 0)),
    ],
    out_specs=pl.BlockSpec((BE, HID), lambda i: (i, 0)),
    out_shape=jax.ShapeDtypeStruct((E, HID), jnp.float32),
)


def _final_body(p, x, woa, wob, bo, o_ref):
    sfull = p[0, :N] + p[1, :N]
    rs = jnp.sum(sfull, axis=1, keepdims=True)
    m = jnp.where(rs == 0.0, x[:], sfull)
    o_ref[:] = jnp.maximum(
        jnp.dot(x[:], woa[:], preferred_element_type=jnp.float32)
        + jnp.dot(m, wob[:], preferred_element_type=jnp.float32) + bo[:], 0.0)


_tc_final = pl.pallas_call(
    _final_body,
    out_shape=jax.ShapeDtypeStruct((N, HID), jnp.float32),
)


def kernel(x, edge_index, edge_attr, rev_edge_index, W_i, b_i, W_h, b_h,
           W_o, b_o):
    src = edge_index[0]
    dst = edge_index[1]
    rev = rev_edge_index
    zeros = jnp.zeros((NP, HID), jnp.float32)
    wia, wib = W_i[:D], W_i[D:]
    woa, wob = W_o[:D], W_o[D:]
    bi = b_i.reshape(1, HID)
    bh = b_h.reshape(1, HID)
    bo = b_o.reshape(1, HID)

    xw = _tc_pre(x, wia)
    xg = _sc_gather(xw, src)
    h0, k = _tc_edge0(xg, edge_attr, wib, bi, W_h)

    ht = None
    for t in range(2):
        p = _sc_scatter(k, dst, zeros)
        s = _tc_combine(p)
        m = _sc_gather_sub(s, k, src, rev)
        if t == 0:
            k = _tc_mid(h0, m, bh, W_h)
        else:
            ht = _tc_last(h0, m, bh)

    p = _sc_scatter(ht, dst, zeros)
    return _tc_final(p, x, woa, wob, bo)


# speedup vs baseline: 1.2747x; 1.0239x over previous
import functools

import jax
import jax.numpy as jnp
from jax import lax
from jax.experimental import pallas as pl
from jax.experimental.pallas import tpu as pltpu
from jax.experimental.pallas import tpu_sc as plsc

N = 10000
E = 320000
D = 128
DE = 16
HID = 128

NC = 2
NS = 16
NW = NC * NS
CW = 80
RW = E // NW
JW = RW // CW
NP = 10240
NTP = NP // NS

BE = 2000
GE = E // BE


def _sc_mesh():
    return plsc.VectorSubcoreMesh(
        core_axis_name="c", subcore_axis_name="s",
        num_cores=NC, num_subcores=NS)


NB = 5


@functools.partial(
    pl.kernel,
    out_type=jax.ShapeDtypeStruct((E, HID), jnp.float32),
    mesh=_sc_mesh(),
    scratch_types=(
        [pltpu.VMEM((RW,), jnp.int32)]
        + [pltpu.VMEM((CW, HID), jnp.float32)] * NB
        + [pltpu.SemaphoreType.DMA] * (2 * NB)
    ),
)
def _sc_gather(tab_hbm, idx_hbm, out_hbm, idx_v, *bufs):
    gb = bufs[:NB]
    sg = bufs[NB:2 * NB]
    sw = bufs[2 * NB:3 * NB]
    w = lax.axis_index("s") * NC + lax.axis_index("c")
    pltpu.sync_copy(idx_hbm.at[pl.ds(w * RW, RW)], idx_v)

    def gath(jj, b):
        off = pl.multiple_of(jj * CW, 16)
        pltpu.async_copy(tab_hbm.at[idx_v.at[pl.ds(off, CW)]], gb[b], sg[b])

    gath(0, 0)

    def step(i, carry):
        for b in range(NB):
            jj = NB * i + b
            nxb = (b + 1) % NB

            @pl.when(jj >= NB - 1)
            def _():
                pltpu.make_async_copy(gb[nxb], out_hbm.at[pl.ds(0, CW)],
                                      sw[nxb]).wait()

            @pl.when(jj < JW - 1)
            def _():
                gath(jj + 1, nxb)

            pltpu.make_async_copy(tab_hbm.at[idx_v.at[pl.ds(0, CW)]], gb[b],
                                  sg[b]).wait()
            pltpu.async_copy(gb[b], out_hbm.at[pl.ds(w * RW + jj * CW, CW)],
                             sw[b])
        return carry

    lax.fori_loop(0, JW // NB, step, 0)
    for b in range(1, NB):
        pltpu.make_async_copy(gb[b], out_hbm.at[pl.ds(0, CW)], sw[b]).wait()


@functools.partial(
    pl.kernel,
    out_type=jax.ShapeDtypeStruct((E, HID), jnp.float32),
    mesh=_sc_mesh(),
    scratch_types=(
        [pltpu.VMEM((RW,), jnp.int32)] * 2
        + [pltpu.VMEM((CW, HID), jnp.float32)] * (2 * NB)
        + [pltpu.SemaphoreType.DMA] * (3 * NB)
    ),
)
def _sc_gather_sub(s_hbm, k_hbm, src_hbm, rev_hbm, out_hbm, isrc_v, irev_v,
                   *bufs):
    gb = bufs[:NB]
    rb = bufs[NB:2 * NB]
    sg = bufs[2 * NB:3 * NB]
    sr = bufs[3 * NB:4 * NB]
    sw = bufs[4 * NB:5 * NB]
    w = lax.axis_index("s") * NC + lax.axis_index("c")
    pltpu.sync_copy(src_hbm.at[pl.ds(w * RW, RW)], isrc_v)
    pltpu.sync_copy(rev_hbm.at[pl.ds(w * RW, RW)], irev_v)

    def gath(jj, b):
        off = pl.multiple_of(jj * CW, 16)
        pltpu.async_copy(s_hbm.at[isrc_v.at[pl.ds(off, CW)]], gb[b], sg[b])
        pltpu.async_copy(k_hbm.at[irev_v.at[pl.ds(off, CW)]], rb[b], sr[b])

    def wait_gath(b):
        pltpu.make_async_copy(s_hbm.at[isrc_v.at[pl.ds(0, CW)]], gb[b],
                              sg[b]).wait()
        pltpu.make_async_copy(k_hbm.at[irev_v.at[pl.ds(0, CW)]], rb[b],
                              sr[b]).wait()

    def sub(b):
        g, r = gb[b], rb[b]

        def row(i, carry):
            for t in range(HID // 16):
                d = pl.ds(t * 16, 16)
                g[i, d] = g[i, d] - r[i, d]
            return carry

        lax.fori_loop(0, CW, row, 0)

    gath(0, 0)

    def step(i, carry):
        for b in range(NB):
            jj = NB * i + b
            nxb = (b + 1) % NB

            @pl.when(jj >= NB - 1)
            def _():
                pltpu.make_async_copy(gb[nxb], out_hbm.at[pl.ds(0, CW)],
                                      sw[nxb]).wait()

            @pl.when(jj < JW - 1)
            def _():
                gath(jj + 1, nxb)

            wait_gath(b)
            sub(b)
            pltpu.async_copy(gb[b], out_hbm.at[pl.ds(w * RW + jj * CW, CW)],
                             sw[b])
        return carry

    lax.fori_loop(0, JW // NB, step, 0)
    for b in range(1, NB):
        pltpu.make_async_copy(gb[b], out_hbm.at[pl.ds(0, CW)], sw[b]).wait()


@functools.partial(
    pl.kernel,
    out_type=jax.ShapeDtypeStruct((NC, NP, HID), jnp.float32),
    mesh=_sc_mesh(),
    scratch_types=[
        pltpu.VMEM((RW,), jnp.int32),
        pltpu.VMEM((CW,), jnp.int32),
        pltpu.VMEM((CW,), jnp.int32),
        pltpu.VMEM((CW, HID), jnp.float32),
        pltpu.VMEM((CW, HID), jnp.float32),
        pltpu.VMEM_SHARED((NP, HID), jnp.float32),
        pltpu.SemaphoreType.DMA,
        pltpu.SemaphoreType.DMA,
    ],
)
def _sc_scatter(rows_hbm, dst_hbm, zero_hbm, out_hbm, idx_v, ci0, ci1,
                rv0, rv1, acc_sh, sl0, sl1):
    c = lax.axis_index("c")
    s = lax.axis_index("s")
    w = s * NC + c
    pltpu.sync_copy(zero_hbm.at[pl.ds(s * NTP, NTP)],
                    acc_sh.at[pl.ds(s * NTP, NTP)])
    pltpu.sync_copy(dst_hbm.at[pl.ds(w * RW, RW)], idx_v)
    plsc.subcore_barrier()
    rv = (rv0, rv1)
    ci = (ci0, ci1)
    sl = (sl0, sl1)

    def load(jj, b):
        pltpu.async_copy(rows_hbm.at[pl.ds(w * RW + jj * CW, CW)], rv[b],
                        sl[b])
        off = pl.multiple_of(jj * CW, 16)
        for t in range(CW // 16):
            ci[b][pl.ds(t * 16, 16)] = idx_v[pl.ds(off + t * 16, 16)]

    def scat(b):
        pltpu.make_async_copy(rows_hbm.at[pl.ds(0, CW)], rv[b], sl[b]).wait()
        pltpu.sync_copy(rv[b], acc_sh.at[ci[b]], add=True)

    load(0, 0)

    def step(i, carry):
        for b in (0, 1):
            jj = 2 * i + b
            load(jj + 1, 1 - b)
            scat(b)
        return carry

    lax.fori_loop(0, (JW - 1) // 2, step, 0)
    scat(0)
    plsc.subcore_barrier()
    pltpu.sync_copy(acc_sh.at[pl.ds(s * NTP, NTP)],
                    out_hbm.at[c, pl.ds(s * NTP, NTP)])


def _pre_body(x_ref, w_ref, o_ref):
    o_ref[:] = jnp.dot(x_ref[:], w_ref[:], preferred_element_type=jnp.float32)


_tc_pre = pl.pallas_call(
    _pre_body,
    out_shape=jax.ShapeDtypeStruct((N, HID), jnp.float32),
)


def _edge0_body(xg, ea, wib, bi, wh, h0_ref, k_ref):
    h0 = xg[:] + jnp.dot(ea[:], wib[:], preferred_element_type=jnp.float32) + bi[:]
    h0_ref[:] = h0.astype(jnp.bfloat16)
    k_ref[:] = jnp.dot(jnp.maximum(h0, 0.0), wh[:],
                       preferred_element_type=jnp.float32)


_tc_edge0 = pl.pallas_call(
    _edge0_body,
    grid=(GE,),
    in_specs=[
        pl.BlockSpec((BE, HID), lambda i: (i, 0)),
        pl.BlockSpec((BE, DE), lambda i: (i, 0)),
        pl.BlockSpec((DE, HID), lambda i: (0, 0)),
        pl.BlockSpec((1, HID), lambda i: (0, 0)),
        pl.BlockSpec((HID, HID), lambda i: (0, 0)),
    ],
    out_specs=[pl.BlockSpec((BE, HID), lambda i: (i, 0))] * 2,
    out_shape=[jax.ShapeDtypeStruct((E, HID), jnp.bfloat16),
               jax.ShapeDtypeStruct((E, HID), jnp.float32)],
)


def _comb_body(p_ref, s_ref):
    s_ref[:] = p_ref[0, :N] + p_ref[1, :N]


_tc_combine = pl.pallas_call(
    _comb_body,
    out_shape=jax.ShapeDtypeStruct((N, HID), jnp.float32),
)


def _mid_body(h0, m, bh, wh, k_ref):
    ht = jnp.maximum(h0[:].astype(jnp.float32) + m[:] + bh[:], 0.0)
    k_ref[:] = jnp.dot(ht, wh[:], preferred_element_type=jnp.float32)


_tc_mid = pl.pallas_call(
    _mid_body,
    grid=(GE,),
    in_specs=[
        pl.BlockSpec((BE, HID), lambda i: (i, 0)),
        pl.BlockSpec((BE, HID), lambda i: (i, 0)),
        pl.BlockSpec((1, HID), lambda i: (0, 0)),
        pl.BlockSpec((HID, HID), lambda i: (0, 0)),
    ],
    out_specs=pl.BlockSpec((BE, HID), lambda i: (i, 0)),
    out_shape=jax.ShapeDtypeStruct((E, HID), jnp.float32),
)


def _last_body(h0, m, bh, ht_ref):
    ht_ref[:] = jnp.maximum(h0[:].astype(jnp.float32) + m[:] + bh[:], 0.0)


_tc_last = pl.pallas_call(
    _last_body,
    grid=(GE,),
    in_specs=[
        pl.BlockSpec((BE, HID), lambda i: (i, 0)),
        pl.BlockSpec((BE, HID), lambda i: (i, 0)),
        pl.BlockSpec((1, HID), lambda i: (0, 0)),
    ],
    out_specs=pl.BlockSpec((BE, HID), lambda i: (i, 0)),
    out_shape=jax.ShapeDtypeStruct((E, HID), jnp.float32),
)


def _final_body(p, x, woa, wob, bo, o_ref):
    sfull = p[0, :N] + p[1, :N]
    rs = jnp.sum(sfull, axis=1, keepdims=True)
    m = jnp.where(rs == 0.0, x[:], sfull)
    o_ref[:] = jnp.maximum(
        jnp.dot(x[:], woa[:], preferred_element_type=jnp.float32)
        + jnp.dot(m, wob[:], preferred_element_type=jnp.float32) + bo[:], 0.0)


_tc_final = pl.pallas_call(
    _final_body,
    out_shape=jax.ShapeDtypeStruct((N, HID), jnp.float32),
)


def kernel(x, edge_index, edge_attr, rev_edge_index, W_i, b_i, W_h, b_h,
           W_o, b_o):
    src = edge_index[0]
    dst = edge_index[1]
    rev = rev_edge_index
    zeros = jnp.zeros((NP, HID), jnp.float32)
    wia, wib = W_i[:D], W_i[D:]
    woa, wob = W_o[:D], W_o[D:]
    bi = b_i.reshape(1, HID)
    bh = b_h.reshape(1, HID)
    bo = b_o.reshape(1, HID)

    xw = _tc_pre(x, wia)
    xg = _sc_gather(xw, src)
    h0, k = _tc_edge0(xg, edge_attr, wib, bi, W_h)

    ht = None
    for t in range(2):
        p = _sc_scatter(k, dst, zeros)
        s = _tc_combine(p)
        m = _sc_gather_sub(s, k, src, rev)
        if t == 0:
            k = _tc_mid(h0, m, bh, W_h)
        else:
            ht = _tc_last(h0, m, bh)

    p = _sc_scatter(ht, dst, zeros)
    return _tc_final(p, x, woa, wob, bo)
